# baseline (speedup 1.0000x reference)
import functools
import jax
import jax.numpy as jnp
from jax import lax
from jax.experimental import pallas as pl
from jax.experimental.pallas import tpu as pltpu
from jax.experimental.pallas import tpu_sc as plsc

T = 512
HA, WA = 181, 360
HB, WB = 91, 180
L = 16
NC, NS = 2, 16
NW = NC * NS

A_NW, A_TAIL = 22, HA - 22 * 8
B_W0, B_NW, B_TAIL = 20, 11, HB - 11 * 8


def _interp_idx(t_vmem, s_vec):
    cnt = jnp.zeros((L,), jnp.int32)
    for j in range(T // L):
        tc = t_vmem[pl.ds(j * L, L)]
        cnt = cnt + plsc.all_reduce_population_count(tc <= s_vec)
    i = jnp.clip(cnt, 1, T - 1)
    t0 = plsc.load_gather(t_vmem, [i - 1])
    t1 = plsc.load_gather(t_vmem, [i])
    f = (i - 1).astype(jnp.float32) + (s_vec - t0) / (t1 - t0)
    f = jnp.where(cnt == 0, jnp.float32(0.0), f)
    f = jnp.where(cnt == T, jnp.float32(T - 1), f)
    r = f.astype(jnp.int32)
    d = f - r.astype(jnp.float32)
    half = jnp.float32(0.5)
    up = (d > half) | ((d == half) & ((r % 2) == 1))
    idx = r + jnp.where(up, 1, 0).astype(jnp.int32)
    return jnp.max(idx)


_mesh = plsc.VectorSubcoreMesh(
    core_axis_name="c", subcore_axis_name="s", num_cores=1, num_subcores=1
)


@functools.partial(
    pl.kernel,
    mesh=_mesh,
    compiler_params=pltpu.CompilerParams(
        needs_layout_passes=False, skip_device_barrier=True
    ),
    out_type=[
        jax.ShapeDtypeStruct((HA, WA), jnp.float32),
        jax.ShapeDtypeStruct((HB, WB), jnp.float32),
    ],
    scratch_types=[
        pltpu.VMEM((T,), jnp.float32),
        pltpu.VMEM((L,), jnp.float32),
        pltpu.SemaphoreType.DMA,
        pltpu.SemaphoreType.DMA,
        pltpu.SemaphoreType.DMA,
    ],
)
def _dyn_slice(ta, da, tb, db, sv, oa, ob, t_vmem, s_vmem, sem_a, sem_b,
               sem_s):
    pltpu.async_copy(sv, s_vmem.at[pl.ds(0, 1)], sem_s).wait()
    s_vec = plsc.load_gather(s_vmem, [jnp.zeros((L,), jnp.int32)])
    pltpu.sync_copy(ta, t_vmem)
    ia = _interp_idx(t_vmem, s_vec)
    ca = pltpu.async_copy(da.at[ia], oa, sem_a)
    pltpu.sync_copy(tb, t_vmem)
    ib = _interp_idx(t_vmem, s_vec)
    cb = pltpu.async_copy(db.at[ib], ob, sem_b)
    ca.wait()
    cb.wait()


def kernel(times_a, data_a, times_b, data_b, sim_time):
    sv = jnp.asarray(sim_time, jnp.float32).reshape(1)
    oa, ob = _dyn_slice(times_a, data_a, times_b, data_b, sv)
    return (oa, ob)


# speedup vs baseline: 1.0041x; 1.0041x over previous
import functools
import jax
import jax.numpy as jnp
from jax import lax
from jax.experimental import pallas as pl
from jax.experimental.pallas import tpu as pltpu
from jax.experimental.pallas import tpu_sc as plsc

T = 512
HA, WA = 181, 360
HB, WB = 91, 180
L = 16


def _interp_idx_vec(t_vmem, s_vec):
    idx = jnp.full((L,), -1, jnp.int32)
    step = T
    while step > 1:
        step //= 2
        probe = idx + step
        tp = plsc.load_gather(t_vmem, [jnp.clip(probe, 0, T - 1)])
        idx = jnp.where(tp <= s_vec, probe, idx)
    n = idx + 1
    i = jnp.clip(n, 1, T - 1)
    t0 = plsc.load_gather(t_vmem, [i - 1])
    t1 = plsc.load_gather(t_vmem, [i])
    f = (i - 1).astype(jnp.float32) + (s_vec - t0) / (t1 - t0)
    f = jnp.where(n == 0, jnp.float32(0.0), f)
    f = jnp.where(n == T, jnp.float32(T - 1), f)
    r = f.astype(jnp.int32)
    d = f - r.astype(jnp.float32)
    half = jnp.float32(0.5)
    up = (d > half) | ((d == half) & ((r % 2) == 1))
    return r + jnp.where(up, 1, 0).astype(jnp.int32)


_mesh = plsc.VectorSubcoreMesh(
    core_axis_name="c", subcore_axis_name="s", num_cores=1, num_subcores=1
)


@functools.partial(
    pl.kernel,
    mesh=_mesh,
    compiler_params=pltpu.CompilerParams(needs_layout_passes=False),
    out_type=jax.ShapeDtypeStruct((2,), jnp.int32),
    scratch_types=[
        pltpu.VMEM((T,), jnp.float32),
        pltpu.VMEM((L,), jnp.float32),
        pltpu.VMEM((L,), jnp.int32),
        pltpu.SemaphoreType.DMA,
    ],
)
def _interp_indices(ta, tb, sv, oi, t_vmem, s_vmem, i_vmem, sem):
    pltpu.async_copy(sv, s_vmem.at[pl.ds(0, 1)], sem).wait()
    s_vec = plsc.load_gather(s_vmem, [jnp.zeros((L,), jnp.int32)])
    pltpu.sync_copy(ta, t_vmem)
    ia_v = _interp_idx_vec(t_vmem, s_vec)
    pltpu.sync_copy(tb, t_vmem)
    ib_v = _interp_idx_vec(t_vmem, s_vec)
    iota = lax.broadcasted_iota(jnp.int32, (L,), 0)
    i_vmem[...] = jnp.where(iota == 0, ia_v, ib_v)
    pltpu.sync_copy(i_vmem.at[pl.ds(0, 2)], oi)


def _slab_copy(idx_ref, da_ref, db_ref, oa_ref, ob_ref, sem_a, sem_b):
    ia = idx_ref[0]
    ib = idx_ref[1]
    ca = pltpu.make_async_copy(da_ref.at[ia], oa_ref, sem_a)
    cb = pltpu.make_async_copy(db_ref.at[ib], ob_ref, sem_b)
    ca.start()
    cb.start()
    ca.wait()
    cb.wait()


_copy_call = pl.pallas_call(
    _slab_copy,
    out_shape=[
        jax.ShapeDtypeStruct((HA, WA), jnp.float32),
        jax.ShapeDtypeStruct((HB, WB), jnp.float32),
    ],
    in_specs=[
        pl.BlockSpec(memory_space=pltpu.SMEM),
        pl.BlockSpec(memory_space=pl.ANY),
        pl.BlockSpec(memory_space=pl.ANY),
    ],
    out_specs=[
        pl.BlockSpec(memory_space=pl.ANY),
        pl.BlockSpec(memory_space=pl.ANY),
    ],
    scratch_shapes=[pltpu.SemaphoreType.DMA, pltpu.SemaphoreType.DMA],
)


def kernel(times_a, data_a, times_b, data_b, sim_time):
    sv = jnp.asarray(sim_time, jnp.float32).reshape(1)
    idx = _interp_indices(times_a, times_b, sv)
    oa, ob = _copy_call(idx, data_a, data_b)
    return (oa, ob)


# speedup vs baseline: 4.2551x; 4.2376x over previous
import functools
import jax
import jax.numpy as jnp
from jax import lax
from jax.experimental import pallas as pl
from jax.experimental.pallas import tpu as pltpu
from jax.experimental.pallas import tpu_sc as plsc

T = 512
HA, WA = 181, 360
HB, WB = 91, 180
L = 16
LANES = 128


def _interp_idx_vec(t_vmem, s_vec):
    n = jnp.zeros((L,), jnp.int32)
    for j in range(T // L):
        tc = t_vmem[pl.ds(j * L, L)]
        n = n + plsc.all_reduce_population_count(tc <= s_vec)
    i = jnp.clip(n, 1, T - 1)
    t0 = plsc.load_gather(t_vmem, [i - 1])
    t1 = plsc.load_gather(t_vmem, [i])
    f = (i - 1).astype(jnp.float32) + (s_vec - t0) / (t1 - t0)
    f = jnp.where(n == 0, jnp.float32(0.0), f)
    f = jnp.where(n == T, jnp.float32(T - 1), f)
    r = f.astype(jnp.int32)
    d = f - r.astype(jnp.float32)
    half = jnp.float32(0.5)
    up = (d > half) | ((d == half) & ((r % 2) == 1))
    return r + jnp.where(up, 1, 0).astype(jnp.int32)


_mesh = plsc.VectorSubcoreMesh(
    core_axis_name="c", subcore_axis_name="s", num_cores=1, num_subcores=1
)


@functools.partial(
    pl.kernel,
    mesh=_mesh,
    compiler_params=pltpu.CompilerParams(needs_layout_passes=False),
    out_type=jax.ShapeDtypeStruct((2, L), jnp.int32),
    scratch_types=[
        pltpu.VMEM((T,), jnp.float32),
        pltpu.VMEM((T,), jnp.float32),
        pltpu.VMEM((L,), jnp.float32),
        pltpu.VMEM((L,), jnp.int32),
        pltpu.VMEM((L,), jnp.int32),
        pltpu.SemaphoreType.DMA,
    ],
)
def _interp_indices(ta, tb, sv, oi, ta_vmem, tb_vmem, s_vmem, ia_vmem,
                    ib_vmem, sem):
    pltpu.async_copy(sv, s_vmem.at[pl.ds(0, 1)], sem).wait()
    s_vec = plsc.load_gather(s_vmem, [jnp.zeros((L,), jnp.int32)])
    pltpu.sync_copy(ta, ta_vmem)
    pltpu.sync_copy(tb, tb_vmem)
    ia_vmem[...] = _interp_idx_vec(ta_vmem, s_vec)
    ib_vmem[...] = _interp_idx_vec(tb_vmem, s_vec)
    pltpu.sync_copy(ia_vmem, oi.at[0])
    pltpu.sync_copy(ib_vmem, oi.at[1])


def _extract(idx_ref, da_ref, db_ref, oa_ref, ob_ref, buf_a, buf_b,
             sem_a, sem_b):
    ia = idx_ref[0, 0]
    ib = idx_ref[1, 0]
    for k in range(T // LANES):
        @pl.when(ia // LANES == k)
        def _():
            pltpu.make_async_copy(
                da_ref.at[:, :, pl.ds(k * LANES, LANES)], buf_a, sem_a
            ).start()

        @pl.when(ib // LANES == k)
        def _():
            pltpu.make_async_copy(
                db_ref.at[:, :, pl.ds(k * LANES, LANES)], buf_b, sem_b
            ).start()

    lane_a = ia % LANES
    lane_b = ib % LANES
    pltpu.make_async_copy(da_ref.at[:, :, pl.ds(0, LANES)], buf_a,
                          sem_a).wait()
    xa = buf_a[...].reshape(HA * WA, LANES)
    oh_a = (lax.broadcasted_iota(jnp.int32, (LANES,), 0) == lane_a
            ).astype(jnp.float32)
    ra = lax.dot_general(xa, oh_a, (((1,), (0,)), ((), ())),
                         preferred_element_type=jnp.float32)
    oa_ref[...] = ra.reshape(HA, WA)

    pltpu.make_async_copy(db_ref.at[:, :, pl.ds(0, LANES)], buf_b,
                          sem_b).wait()
    xb = buf_b[...].reshape(HB * WB, LANES)
    oh_b = (lax.broadcasted_iota(jnp.int32, (LANES,), 0) == lane_b
            ).astype(jnp.float32)
    rb = lax.dot_general(xb, oh_b, (((1,), (0,)), ((), ())),
                         preferred_element_type=jnp.float32)
    ob_ref[...] = rb.reshape(HB, WB)


_extract_call = pl.pallas_call(
    _extract,
    out_shape=[
        jax.ShapeDtypeStruct((HA, WA), jnp.float32),
        jax.ShapeDtypeStruct((HB, WB), jnp.float32),
    ],
    in_specs=[
        pl.BlockSpec(memory_space=pltpu.SMEM),
        pl.BlockSpec(memory_space=pl.ANY),
        pl.BlockSpec(memory_space=pl.ANY),
    ],
    out_specs=[
        pl.BlockSpec(memory_space=pltpu.VMEM),
        pl.BlockSpec(memory_space=pltpu.VMEM),
    ],
    scratch_shapes=[
        pltpu.VMEM((HA, WA, LANES), jnp.float32),
        pltpu.VMEM((HB, WB, LANES), jnp.float32),
        pltpu.SemaphoreType.DMA,
        pltpu.SemaphoreType.DMA,
    ],
    compiler_params=pltpu.CompilerParams(vmem_limit_bytes=50 * 1024 * 1024),
)


def kernel(times_a, data_a, times_b, data_b, sim_time):
    sv = jnp.asarray(sim_time, jnp.float32).reshape(1)
    idx = _interp_indices(times_a, times_b, sv)
    oa, ob = _extract_call(
        idx, data_a.transpose(1, 2, 0), data_b.transpose(1, 2, 0)
    )
    return (oa, ob)
